# baseline (speedup 1.0000x reference)
import jax
import jax.numpy as jnp
from jax.experimental import pallas as pl
from jax.experimental.pallas import tpu as pltpu


def _pool_kernel(ht_ref, w_ref, o_ref):
    t = jnp.tanh(ht_ref[...])
    prod = t * w_ref[...][:, :, None]
    scores = jnp.sum(prod, axis=1)
    m = jnp.max(scores, axis=-1, keepdims=True)
    e = jnp.exp(scores - m)
    den = jnp.sum(e, axis=-1, keepdims=True)
    o_ref[...] = (e / den)[:, None, :]


def kernel(H, weight, bias):
    B, S, D = H.shape
    del bias
    Ht = jnp.transpose(H, (0, 2, 1))
    w32 = weight.reshape(1, D).astype(jnp.float32)

    TB = min(B, 128)
    while B % TB:
        TB //= 2
    grid = (pl.cdiv(B, TB),)
    out = pl.pallas_call(
        _pool_kernel,
        out_shape=jax.ShapeDtypeStruct((B, 1, S), H.dtype),
        grid=grid,
        in_specs=[
            pl.BlockSpec((TB, D, S), lambda b: (b, 0, 0)),
            pl.BlockSpec((1, D), lambda b: (0, 0)),
        ],
        out_specs=pl.BlockSpec((TB, 1, S), lambda b: (b, 0, 0)),
        compiler_params=pltpu.CompilerParams(
            dimension_semantics=("parallel",),
            vmem_limit_bytes=64 << 20,
        ),
    )(Ht, w32)
    return out


# speedup vs baseline: 1.1454x; 1.1454x over previous
import jax
import jax.numpy as jnp
from jax.experimental import pallas as pl
from jax.experimental.pallas import tpu as pltpu


def _pool_kernel(ht_ref, w_ref, o_ref):
    t = jnp.tanh(ht_ref[...])
    prod = t * w_ref[...][:, :, None]
    scores = jnp.sum(prod, axis=1)
    m = jnp.max(scores, axis=-1, keepdims=True)
    e = jnp.exp(scores - m)
    den = jnp.sum(e, axis=-1, keepdims=True)
    o_ref[...] = (e / den)[:, None, :]


def kernel(H, weight, bias):
    B, S, D = H.shape
    del bias
    Ht = jnp.transpose(H, (0, 2, 1))
    w32 = weight.reshape(1, D).astype(jnp.float32)

    TB = min(B, 256)
    while B % TB:
        TB //= 2
    grid = (pl.cdiv(B, TB),)
    out = pl.pallas_call(
        _pool_kernel,
        out_shape=jax.ShapeDtypeStruct((B, 1, S), H.dtype),
        grid=grid,
        in_specs=[
            pl.BlockSpec((TB, D, S), lambda b: (b, 0, 0)),
            pl.BlockSpec((1, D), lambda b: (0, 0)),
        ],
        out_specs=pl.BlockSpec((TB, 1, S), lambda b: (b, 0, 0)),
        compiler_params=pltpu.CompilerParams(
            dimension_semantics=("parallel",),
            vmem_limit_bytes=64 << 20,
        ),
    )(Ht, w32)
    return out
